# Initial kernel scaffold; baseline (speedup 1.0000x reference)
import jax
import jax.numpy as jnp
from jax.experimental import pallas as pl


def kernel(edge_index, edge_attrs, table):
    raise NotImplementedError("write your pallas kernel here")


# speedup vs baseline: 15.5443x; 15.5443x over previous
import functools

import jax
import jax.numpy as jnp
from jax import lax
from jax.experimental import pallas as pl
from jax.experimental.pallas import tpu as pltpu
from jax.experimental.pallas import tpu_sc as plsc

N = 50000
D = 64
E = 800000
HALF = 32
NS = 16
NC = 2

CH = 512
CHD = 1024
EPAD = 819200
EPT = EPAD // NS
NCH = EPT // CH
EPT32 = EPAD // (NS * NC)
NCH32 = EPT32 // CHD
NPAD = 50560
RPT = NPAD // NS
NDUMMY = 512


def _wb_chunks(chunk, total):
    return tuple((o, min(chunk, total - o)) for o in range(0, total, chunk))

_mesh = functools.partial(
    plsc.VectorSubcoreMesh, core_axis_name="c", subcore_axis_name="s")

_SC_PARAMS = pltpu.CompilerParams(use_tc_tiling_on_sc=False)


@functools.partial(
    pl.kernel,
    mesh=_mesh(),
    out_type=jax.ShapeDtypeStruct((NC * NPAD,), jnp.float32),
    compiler_params=_SC_PARAMS,
    scratch_types=[
        pltpu.VMEM((CHD,), jnp.int32),
        pltpu.VMEM((CHD,), jnp.float32),
        pltpu.VMEM_SHARED((NPAD,), jnp.float32),
    ],
)
def _deg_call(dst_hbm, out_hbm, dstb, ones, acc):
    c = lax.axis_index("c")
    s = lax.axis_index("s")

    def _fill(i, val):
        ones[pl.ds(i * 16, 16)] = jnp.full((16,), val, jnp.float32)
        return val

    lax.fori_loop(0, CHD // 16, _fill, 0.0)
    for off, sz in _wb_chunks(CHD, RPT):
        pltpu.sync_copy(ones.at[pl.ds(0, sz)], acc.at[pl.ds(s * RPT + off, sz)])
    lax.fori_loop(0, CHD // 16, _fill, 1.0)
    plsc.subcore_barrier()

    wid = s * NC + c

    def _chunk(i, _):
        base = wid * EPT32 + i * CHD
        pltpu.sync_copy(dst_hbm.at[pl.ds(base, CHD)], dstb)
        pltpu.sync_copy(ones, acc.at[dstb], add=True)
        return 0

    lax.fori_loop(0, NCH32, _chunk, 0)
    plsc.subcore_barrier()

    for off, sz in _wb_chunks(CHD, RPT):
        r0 = s * RPT + off
        pltpu.sync_copy(acc.at[pl.ds(r0, sz)], ones.at[pl.ds(0, sz)])
        pltpu.sync_copy(ones.at[pl.ds(0, sz)], out_hbm.at[pl.ds(c * NPAD + r0, sz)])


@functools.partial(
    pl.kernel,
    mesh=_mesh(),
    out_type=jax.ShapeDtypeStruct((NC * NPAD, HALF), jnp.float32),
    compiler_params=_SC_PARAMS,
    scratch_types=[
        pltpu.VMEM((CH,), jnp.int32),
        pltpu.VMEM((CH,), jnp.int32),
        pltpu.VMEM((CH, HALF), jnp.float32),
        pltpu.VMEM_SHARED((NPAD, HALF), jnp.float32),
        pltpu.SemaphoreType.DMA,
    ],
)
def _layer_call(y_hbm, src_hbm, dst_hbm, out_hbm, srcb, dstb, msg, acc, sem):
    c = lax.axis_index("c")
    s = lax.axis_index("s")

    def _zero(i, _):
        msg[i >> 1, pl.ds((i & 1) * 16, 16)] = jnp.zeros((16,), jnp.float32)
        return 0

    lax.fori_loop(0, CH * 2, _zero, 0)
    for off, sz in _wb_chunks(CH, RPT):
        pltpu.sync_copy(msg.at[pl.ds(0, sz)], acc.at[pl.ds(s * RPT + off, sz)])
    plsc.subcore_barrier()

    def _chunk(i, _):
        base = s * EPT + i * CH
        pltpu.sync_copy(src_hbm.at[pl.ds(c * EPAD + base, CH)], srcb)
        pltpu.sync_copy(dst_hbm.at[pl.ds(base, CH)], dstb)
        pltpu.async_copy(y_hbm.at[srcb], msg, sem).wait()
        pltpu.sync_copy(msg, acc.at[dstb], add=True)
        return 0

    lax.fori_loop(0, NCH, _chunk, 0)
    plsc.subcore_barrier()

    for off, sz in _wb_chunks(CH, RPT):
        r0 = s * RPT + off
        pltpu.sync_copy(acc.at[pl.ds(r0, sz)], msg.at[pl.ds(0, sz)])
        pltpu.sync_copy(msg.at[pl.ds(0, sz)], out_hbm.at[pl.ds(c * NPAD + r0, sz)])


def kernel(edge_index, edge_attrs, table):
    del edge_attrs
    src = edge_index[0]
    dst = edge_index[1]

    pad_i = jnp.arange(EPAD - E, dtype=jnp.int32)
    src_p = jnp.concatenate([src, pad_i % N])
    dst_p = jnp.concatenate([dst, N + pad_i % NDUMMY])
    src2 = jnp.concatenate([src_p, src_p + N])

    degp = _deg_call(dst_p)
    deg = degp[:N] + degp[NPAD:NPAD + N]
    deg_s = jnp.sqrt(deg)
    dinv = jnp.where(deg_s > 0, 1.0 / jnp.maximum(deg_s, 1e-12), 0.0)
    d3 = dinv[None, :, None]

    tsplit = table.reshape(N, NC, HALF).transpose(1, 0, 2)
    y = (d3 * tsplit).reshape(NC * N, HALF)
    acc = tsplit
    for layer in range(3):
        raw = _layer_call(y, src2, dst_p)
        raw = raw.reshape(NC, NPAD, HALF)[:, :N, :]
        acc = acc + d3 * raw
        if layer < 2:
            y = (d3 * d3 * raw).reshape(NC * N, HALF)

    out = (acc * 0.25).transpose(1, 0, 2).reshape(N, D)
    return (table, out)


# speedup vs baseline: 20.7497x; 1.3349x over previous
import functools

import jax
import jax.numpy as jnp
from jax import lax
from jax.experimental import pallas as pl
from jax.experimental.pallas import tpu as pltpu
from jax.experimental.pallas import tpu_sc as plsc

N = 50000
D = 64
E = 800000
HALF = 32
NS = 16
NC = 2

CH = 320
CHD = 1024
EPAD = 819200
EPT = EPAD // NS
NCH = EPT // CH
EPT32 = EPAD // (NS * NC)
NCH32 = EPT32 // CHD
NPAD = 50560
RPT = NPAD // NS
NDUMMY = 512


def _wb_chunks(chunk, total):
    return tuple((o, min(chunk, total - o)) for o in range(0, total, chunk))

_mesh = functools.partial(
    plsc.VectorSubcoreMesh, core_axis_name="c", subcore_axis_name="s")

_SC_PARAMS = pltpu.CompilerParams(use_tc_tiling_on_sc=False)


@functools.partial(
    pl.kernel,
    mesh=_mesh(),
    out_type=jax.ShapeDtypeStruct((NC * NPAD,), jnp.float32),
    compiler_params=_SC_PARAMS,
    scratch_types=[
        pltpu.VMEM((CHD,), jnp.int32),
        pltpu.VMEM((CHD,), jnp.float32),
        pltpu.VMEM_SHARED((NPAD,), jnp.float32),
    ],
)
def _deg_call(dst_hbm, out_hbm, dstb, ones, acc):
    c = lax.axis_index("c")
    s = lax.axis_index("s")

    def _fill(i, val):
        ones[pl.ds(i * 16, 16)] = jnp.full((16,), val, jnp.float32)
        return val

    lax.fori_loop(0, CHD // 16, _fill, 0.0)
    for off, sz in _wb_chunks(CHD, RPT):
        pltpu.sync_copy(ones.at[pl.ds(0, sz)], acc.at[pl.ds(s * RPT + off, sz)])
    lax.fori_loop(0, CHD // 16, _fill, 1.0)
    plsc.subcore_barrier()

    wid = s * NC + c

    def _chunk(i, _):
        base = wid * EPT32 + i * CHD
        pltpu.sync_copy(dst_hbm.at[pl.ds(base, CHD)], dstb)
        pltpu.sync_copy(ones, acc.at[dstb], add=True)
        return 0

    lax.fori_loop(0, NCH32, _chunk, 0)
    plsc.subcore_barrier()

    for off, sz in _wb_chunks(CHD, RPT):
        r0 = s * RPT + off
        pltpu.sync_copy(acc.at[pl.ds(r0, sz)], ones.at[pl.ds(0, sz)])
        pltpu.sync_copy(ones.at[pl.ds(0, sz)], out_hbm.at[pl.ds(c * NPAD + r0, sz)])


@functools.partial(
    pl.kernel,
    mesh=_mesh(),
    out_type=jax.ShapeDtypeStruct((NC * NPAD, HALF), jnp.float32),
    compiler_params=_SC_PARAMS,
    scratch_types=[
        pltpu.VMEM((CH,), jnp.int32),
        pltpu.VMEM((CH,), jnp.int32),
        pltpu.VMEM((CH,), jnp.int32),
        pltpu.VMEM((CH,), jnp.int32),
        pltpu.VMEM((CH, HALF), jnp.float32),
        pltpu.VMEM((CH, HALF), jnp.float32),
        pltpu.VMEM_SHARED((NPAD, HALF), jnp.float32),
        pltpu.SemaphoreType.DMA,
        pltpu.SemaphoreType.DMA,
        pltpu.SemaphoreType.DMA,
        pltpu.SemaphoreType.DMA,
        pltpu.SemaphoreType.DMA,
        pltpu.SemaphoreType.DMA,
    ],
)
def _layer_call(y_hbm, src_hbm, dst_hbm, out_hbm,
                srcb0, srcb1, dstb0, dstb1, msg0, msg1, acc,
                isem0, isem1, gsem0, gsem1, ssem0, ssem1):
    c = lax.axis_index("c")
    s = lax.axis_index("s")
    srcb = (srcb0, srcb1)
    dstb = (dstb0, dstb1)
    msg = (msg0, msg1)
    isem = (isem0, isem1)
    gsem = (gsem0, gsem1)
    ssem = (ssem0, ssem1)

    def _zero(i, _):
        msg0[i >> 1, pl.ds((i & 1) * 16, 16)] = jnp.zeros((16,), jnp.float32)
        msg1[i >> 1, pl.ds((i & 1) * 16, 16)] = jnp.zeros((16,), jnp.float32)
        return 0

    lax.fori_loop(0, CH * 2, _zero, 0)
    for k, (off, sz) in enumerate(_wb_chunks(CH, RPT)):
        pltpu.async_copy(msg[k % 2].at[pl.ds(0, sz)],
                         acc.at[pl.ds(s * RPT + off, sz)], gsem[k % 2])
    for k, (off, sz) in enumerate(_wb_chunks(CH, RPT)):
        pltpu.make_async_copy(msg[k % 2].at[pl.ds(0, sz)],
                              acc.at[pl.ds(s * RPT + off, sz)], gsem[k % 2]).wait()
    plsc.subcore_barrier()

    def _issue_front(i, b, wait_prev_scatter):
        if wait_prev_scatter:
            pltpu.make_async_copy(msg[b], acc.at[dstb[b]], ssem[b]).wait()
        base = s * EPT + i * CH
        c1 = pltpu.async_copy(src_hbm.at[pl.ds(c * EPAD + base, CH)],
                              srcb[b], isem[b])
        c2 = pltpu.async_copy(dst_hbm.at[pl.ds(base, CH)], dstb[b], isem[b])
        c1.wait()
        c2.wait()
        pltpu.async_copy(y_hbm.at[srcb[b]], msg[b], gsem[b])

    def _issue_back(b):
        pltpu.make_async_copy(y_hbm.at[srcb[b]], msg[b], gsem[b]).wait()
        pltpu.async_copy(msg[b], acc.at[dstb[b]], ssem[b], add=True)

    _issue_front(0, 0, False)
    _issue_front(1, 1, False)
    _issue_back(0)
    _issue_back(1)

    def _pair(g, _):
        _issue_front(2 * g, 0, True)
        _issue_front(2 * g + 1, 1, True)
        _issue_back(0)
        _issue_back(1)
        return 0

    lax.fori_loop(1, NCH // 2, _pair, 0)
    pltpu.make_async_copy(msg0, acc.at[dstb0], ssem0).wait()
    pltpu.make_async_copy(msg1, acc.at[dstb1], ssem1).wait()
    plsc.subcore_barrier()

    wtiles = _wb_chunks(CH, RPT)
    for k, (off, sz) in enumerate(wtiles):
        b = k % 2
        r0 = s * RPT + off
        if k >= 2:
            pltpu.make_async_copy(
                msg[b].at[pl.ds(0, wtiles[k - 2][1])],
                out_hbm.at[pl.ds(c * NPAD + s * RPT + wtiles[k - 2][0],
                                 wtiles[k - 2][1])], ssem[b]).wait()
        pltpu.async_copy(acc.at[pl.ds(r0, sz)], msg[b].at[pl.ds(0, sz)],
                         gsem[b]).wait()
        pltpu.async_copy(msg[b].at[pl.ds(0, sz)],
                         out_hbm.at[pl.ds(c * NPAD + r0, sz)], ssem[b])
    for k in (len(wtiles) - 2, len(wtiles) - 1):
        off, sz = wtiles[k]
        pltpu.make_async_copy(
            msg[k % 2].at[pl.ds(0, sz)],
            out_hbm.at[pl.ds(c * NPAD + s * RPT + off, sz)], ssem[k % 2]).wait()


def kernel(edge_index, edge_attrs, table):
    del edge_attrs
    src = edge_index[0]
    dst = edge_index[1]

    pad_i = jnp.arange(EPAD - E, dtype=jnp.int32)
    src_p = jnp.concatenate([src, pad_i % N])
    dst_p = jnp.concatenate([dst, N + pad_i % NDUMMY])
    src2 = jnp.concatenate([src_p, src_p + N])

    degp = _deg_call(dst_p)
    deg = degp[:N] + degp[NPAD:NPAD + N]
    deg_s = jnp.sqrt(deg)
    dinv = jnp.where(deg_s > 0, 1.0 / jnp.maximum(deg_s, 1e-12), 0.0)
    d3 = dinv[None, :, None]

    tsplit = table.reshape(N, NC, HALF).transpose(1, 0, 2)
    y = (d3 * tsplit).reshape(NC * N, HALF)
    acc = tsplit
    for layer in range(3):
        raw = _layer_call(y, src2, dst_p)
        raw = raw.reshape(NC, NPAD, HALF)[:, :N, :]
        acc = acc + d3 * raw
        if layer < 2:
            y = (d3 * d3 * raw).reshape(NC * N, HALF)

    out = (acc * 0.25).transpose(1, 0, 2).reshape(N, D)
    return (table, out)
